# baseline (speedup 1.0000x reference)
import functools
import math

import jax
import jax.numpy as jnp
from jax import lax
from jax.experimental import pallas as pl
from jax.experimental.pallas import tpu as pltpu
from jax.experimental.pallas import tpu_sc as plsc

C = 96
H = 48
W = 48
P = H * W
WIN = 13
HALF = WIN // 2
K2 = WIN * WIN
K2P = 176
KSEL = 8
NEG_INF = float("-inf")

NWORK = 18
PXW = P // NWORK
LANES = 16


def _sim_kernel(x_ref, sim_ref):
    X = x_ref[...]
    n2 = jnp.sum(X * X, axis=0, keepdims=True)
    norm = jnp.maximum(jnp.sqrt(n2), 1e-12)
    XN = X / norm

    p = lax.broadcasted_iota(jnp.int32, (1, P), 1)
    px = p % W
    py = p // W

    def sim_body(k, _):
        dy = k // WIN - HALF
        dx = k % WIN - HALF
        off = dy * W + dx
        sh = pltpu.roll(XN, jnp.mod(-off, P), axis=1)
        s = jnp.sum(XN * sh, axis=0, keepdims=True)
        nx = px + dx
        ny = py + dy
        valid = (nx >= 0) & (nx < W) & (ny >= 0) & (ny < H)
        sim_ref[pl.ds(k, 1), :] = jnp.where(valid, s, NEG_INF)
        return 0

    lax.fori_loop(0, K2, sim_body, 0)
    sim_ref[K2:, :] = jnp.full((K2P - K2, P), NEG_INF, jnp.float32)


def _topk_kernel(sim_hbm, out_hbm, sblk, obuf, sem):
    wid = lax.axis_index("s") * 2 + lax.axis_index("c")

    @pl.when(wid < NWORK)
    def _():
        base = wid * PXW
        pltpu.sync_copy(sim_hbm.at[:, pl.ds(base, PXW)], sblk)

        for g in range(PXW // LANES):
            lo = g * LANES

            def body(k, t):
                v = sblk[k, pl.ds(lo, LANES)]
                t1, t2, t3, t4, t5, t6, t7, t8 = t
                n8 = jnp.maximum(t8, jnp.minimum(t7, v))
                n7 = jnp.maximum(t7, jnp.minimum(t6, v))
                n6 = jnp.maximum(t6, jnp.minimum(t5, v))
                n5 = jnp.maximum(t5, jnp.minimum(t4, v))
                n4 = jnp.maximum(t4, jnp.minimum(t3, v))
                n3 = jnp.maximum(t3, jnp.minimum(t2, v))
                n2 = jnp.maximum(t2, jnp.minimum(t1, v))
                n1 = jnp.maximum(t1, v)
                return (n1, n2, n3, n4, n5, n6, n7, n8)

            init = tuple(jnp.full((LANES,), NEG_INF, jnp.float32)
                         for _ in range(KSEL))
            t = lax.fori_loop(0, K2P, body, init)
            den = (jnp.exp(t[0]) + jnp.exp(t[1]) + jnp.exp(t[2])
                   + jnp.exp(t[3]) + jnp.exp(t[4]) + jnp.exp(t[5])
                   + jnp.exp(t[6]) + jnp.exp(t[7]))
            obuf[0, pl.ds(lo, LANES)] = t[7]
            obuf[1, pl.ds(lo, LANES)] = den

        pltpu.sync_copy(obuf, out_hbm.at[:, pl.ds(base, PXW)])


def _topk_call(sim):
    mesh = plsc.VectorSubcoreMesh(core_axis_name="c", subcore_axis_name="s")
    return pl.kernel(
        _topk_kernel,
        out_type=jax.ShapeDtypeStruct((2, P), jnp.float32),
        mesh=mesh,
        scratch_types=[
            pltpu.VMEM((K2P, PXW), jnp.float32),
            pltpu.VMEM((2, PXW), jnp.float32),
            pltpu.SemaphoreType.DMA,
        ],
    )(sim)


def _agg_kernel(x_ref, sim_ref, cd_ref, lnw_ref, lnb_ref, w1_ref, b1_ref,
                w2_ref, b2_ref, out_ref, agg_ref):
    X = x_ref[...]
    cut = cd_ref[0:1, :]
    den = cd_ref[1:2, :]

    agg_ref[...] = jnp.zeros((C, P), jnp.float32)

    def agg_body(k, _):
        dy = k // WIN - HALF
        dx = k % WIN - HALF
        off = dy * W + dx
        srow = sim_ref[pl.ds(k, 1), :]
        wgt = jnp.where(srow >= cut, jnp.exp(srow), 0.0)
        xsh = pltpu.roll(X, jnp.mod(-off, P), axis=1)
        agg_ref[...] += wgt * xsh
        return 0

    lax.fori_loop(0, K2, agg_body, 0)
    agg = agg_ref[...] / den

    mu = jnp.sum(X, axis=0, keepdims=True) * (1.0 / C)
    xc = X - mu
    var = jnp.sum(xc * xc, axis=0, keepdims=True) * (1.0 / C)
    xln = xc / jnp.sqrt(var + 1e-5) * lnw_ref[...] + lnb_ref[...]

    enh = agg + xln

    h = jnp.dot(w1_ref[...], enh, preferred_element_type=jnp.float32)
    h = jnp.maximum(h + b1_ref[...], 0.0)
    ffn = jnp.dot(w2_ref[...], h, preferred_element_type=jnp.float32)
    out_ref[...] = enh + ffn + b2_ref[...]


@jax.jit
def kernel(x, ln_w, ln_b, w1, b1, w2, b2):
    xf = x.reshape(C, P)
    sim = pl.pallas_call(
        _sim_kernel,
        out_shape=jax.ShapeDtypeStruct((K2P, P), jnp.float32),
    )(xf)
    cutden = _topk_call(sim)
    out = pl.pallas_call(
        _agg_kernel,
        out_shape=jax.ShapeDtypeStruct((C, P), jnp.float32),
        scratch_shapes=[
            pltpu.VMEM((C, P), jnp.float32),
        ],
    )(xf, sim, cutden, ln_w.reshape(C, 1), ln_b.reshape(C, 1),
      w1, b1.reshape(2 * C, 1), w2, b2.reshape(C, 1))
    return out.reshape(1, C, H, W)


# speedup vs baseline: 1.8648x; 1.8648x over previous
import functools
import math

import jax
import jax.numpy as jnp
from jax import lax
from jax.experimental import pallas as pl
from jax.experimental.pallas import tpu as pltpu
from jax.experimental.pallas import tpu_sc as plsc

C = 96
H = 48
W = 48
P = H * W
WIN = 13
HALF = WIN // 2
K2 = WIN * WIN
K2P = 176
KSEL = 8
NEG_INF = float("-inf")

NWORK = 18
PXW = P // NWORK
LANES = 16


def _sim_kernel(x_ref, sim_ref):
    X = x_ref[...]
    n2 = jnp.sum(X * X, axis=0, keepdims=True)
    norm = jnp.maximum(jnp.sqrt(n2), 1e-12)
    XN = X / norm

    p = lax.broadcasted_iota(jnp.int32, (1, P), 1)
    px = p % W
    py = p // W

    def valid_mask(dy, dx):
        nx = px + dx
        ny = py + dy
        return (nx >= 0) & (nx < W) & (ny >= 0) & (ny < H)

    def emit_pair(xn_dy, dy, dx, k):
        sh = pltpu.roll(xn_dy, (-dx) % P, axis=1)
        s_raw = jnp.sum(XN * sh, axis=0, keepdims=True)
        sim_ref[pl.ds(k, 1), :] = jnp.where(valid_mask(dy, dx), s_raw,
                                            NEG_INF)
        if isinstance(k, int) and k == K2 // 2:
            return
        off = dy * W + dx
        s_neg = pltpu.roll(s_raw, jnp.mod(off, P), axis=1)
        sim_ref[pl.ds(K2 - 1 - k, 1), :] = jnp.where(valid_mask(-dy, -dx),
                                                     s_neg, NEG_INF)

    for dx in range(0, HALF + 1):
        emit_pair(XN, 0, dx, (K2 // 2) + dx)

    def dy_body(dy, _):
        xn_dy = pltpu.roll(XN, jnp.mod(-dy * W, P), axis=1)
        for dx in range(-HALF, HALF + 1):
            k = (dy + HALF) * WIN + (dx + HALF)
            emit_pair(xn_dy, dy, dx, k)
        return 0

    lax.fori_loop(1, HALF + 1, dy_body, 0)
    sim_ref[K2:, :] = jnp.full((K2P - K2, P), NEG_INF, jnp.float32)


def _topk_kernel(sim_hbm, out_hbm, sblk, obuf, sem):
    wid = lax.axis_index("s") * 2 + lax.axis_index("c")

    @pl.when(wid < NWORK)
    def _():
        base = wid * PXW
        pltpu.sync_copy(sim_hbm.at[:, pl.ds(base, PXW)], sblk)

        for g in range(PXW // LANES):
            lo = g * LANES

            def body(k, t):
                v = sblk[k, pl.ds(lo, LANES)]
                t1, t2, t3, t4, t5, t6, t7, t8 = t
                n8 = jnp.maximum(t8, jnp.minimum(t7, v))
                n7 = jnp.maximum(t7, jnp.minimum(t6, v))
                n6 = jnp.maximum(t6, jnp.minimum(t5, v))
                n5 = jnp.maximum(t5, jnp.minimum(t4, v))
                n4 = jnp.maximum(t4, jnp.minimum(t3, v))
                n3 = jnp.maximum(t3, jnp.minimum(t2, v))
                n2 = jnp.maximum(t2, jnp.minimum(t1, v))
                n1 = jnp.maximum(t1, v)
                return (n1, n2, n3, n4, n5, n6, n7, n8)

            init = tuple(jnp.full((LANES,), NEG_INF, jnp.float32)
                         for _ in range(KSEL))
            t = lax.fori_loop(0, K2P, body, init)
            den = (jnp.exp(t[0]) + jnp.exp(t[1]) + jnp.exp(t[2])
                   + jnp.exp(t[3]) + jnp.exp(t[4]) + jnp.exp(t[5])
                   + jnp.exp(t[6]) + jnp.exp(t[7]))
            obuf[0, pl.ds(lo, LANES)] = t[7]
            obuf[1, pl.ds(lo, LANES)] = den

        pltpu.sync_copy(obuf, out_hbm.at[:, pl.ds(base, PXW)])


def _topk_call(sim):
    mesh = plsc.VectorSubcoreMesh(core_axis_name="c", subcore_axis_name="s")
    return pl.kernel(
        _topk_kernel,
        out_type=jax.ShapeDtypeStruct((2, P), jnp.float32),
        mesh=mesh,
        scratch_types=[
            pltpu.VMEM((K2P, PXW), jnp.float32),
            pltpu.VMEM((2, PXW), jnp.float32),
            pltpu.SemaphoreType.DMA,
        ],
    )(sim)


def _agg_kernel(x_ref, sim_ref, cd_ref, lnw_ref, lnb_ref, w1_ref, b1_ref,
                w2_ref, b2_ref, out_ref, agg_ref):
    X = x_ref[...]
    cut = cd_ref[0:1, :]
    den = cd_ref[1:2, :]

    agg_ref[...] = jnp.zeros((C, P), jnp.float32)

    def agg_body(dy, _):
        x_dy = pltpu.roll(X, jnp.mod(-(dy - HALF) * W, P), axis=1)
        subtotal = jnp.zeros((C, P), jnp.float32)
        for dx in range(-HALF, HALF + 1):
            k = dy * WIN + (dx + HALF)
            srow = sim_ref[pl.ds(k, 1), :]
            wgt = jnp.where(srow >= cut, jnp.exp(srow), 0.0)
            xsh = pltpu.roll(x_dy, (-dx) % P, axis=1)
            subtotal = subtotal + wgt * xsh
        agg_ref[...] += subtotal
        return 0

    lax.fori_loop(0, WIN, agg_body, 0)
    agg = agg_ref[...] / den

    mu = jnp.sum(X, axis=0, keepdims=True) * (1.0 / C)
    xc = X - mu
    var = jnp.sum(xc * xc, axis=0, keepdims=True) * (1.0 / C)
    xln = xc / jnp.sqrt(var + 1e-5) * lnw_ref[...] + lnb_ref[...]

    enh = agg + xln

    h = jnp.dot(w1_ref[...], enh, preferred_element_type=jnp.float32)
    h = jnp.maximum(h + b1_ref[...], 0.0)
    ffn = jnp.dot(w2_ref[...], h, preferred_element_type=jnp.float32)
    out_ref[...] = enh + ffn + b2_ref[...]


@jax.jit
def kernel(x, ln_w, ln_b, w1, b1, w2, b2):
    xf = x.reshape(C, P)
    sim = pl.pallas_call(
        _sim_kernel,
        out_shape=jax.ShapeDtypeStruct((K2P, P), jnp.float32),
    )(xf)
    cutden = _topk_call(sim)
    out = pl.pallas_call(
        _agg_kernel,
        out_shape=jax.ShapeDtypeStruct((C, P), jnp.float32),
        scratch_shapes=[
            pltpu.VMEM((C, P), jnp.float32),
        ],
    )(xf, sim, cutden, ln_w.reshape(C, 1), ln_b.reshape(C, 1),
      w1, b1.reshape(2 * C, 1), w2, b2.reshape(C, 1))
    return out.reshape(1, C, H, W)


# speedup vs baseline: 2.2845x; 1.2251x over previous
import functools
import math

import jax
import jax.numpy as jnp
from jax import lax
from jax.experimental import pallas as pl
from jax.experimental.pallas import tpu as pltpu
from jax.experimental.pallas import tpu_sc as plsc

C = 96
H = 48
W = 48
P = H * W
WIN = 13
HALF = WIN // 2
K2 = WIN * WIN
K2P = 176
KSEL = 8
NEG_INF = float("-inf")

NWORK = 18
PXW = P // NWORK
LANES = 16
NGRP = PXW // LANES
CHUNK = PXW // 2 * KSEL


def _sim_kernel(x_ref, sim_ref):
    X = x_ref[...]
    n2 = jnp.sum(X * X, axis=0, keepdims=True)
    norm = jnp.maximum(jnp.sqrt(n2), 1e-12)
    XN = X / norm

    p = lax.broadcasted_iota(jnp.int32, (1, P), 1)
    px = p % W
    py = p // W

    def valid_mask(dy, dx):
        nx = px + dx
        ny = py + dy
        return (nx >= 0) & (nx < W) & (ny >= 0) & (ny < H)

    def emit_pair(xn_dy, dy, dx, k):
        sh = pltpu.roll(xn_dy, (-dx) % P, axis=1)
        s_raw = jnp.sum(XN * sh, axis=0, keepdims=True)
        sim_ref[pl.ds(k, 1), :] = jnp.where(valid_mask(dy, dx), s_raw,
                                            NEG_INF)
        if isinstance(k, int) and k == K2 // 2:
            return
        off = dy * W + dx
        s_neg = pltpu.roll(s_raw, jnp.mod(off, P), axis=1)
        sim_ref[pl.ds(K2 - 1 - k, 1), :] = jnp.where(valid_mask(-dy, -dx),
                                                     s_neg, NEG_INF)

    for dx in range(0, HALF + 1):
        emit_pair(XN, 0, dx, (K2 // 2) + dx)

    def dy_body(dy, _):
        xn_dy = pltpu.roll(XN, jnp.mod(-dy * W, P), axis=1)
        for dx in range(-HALF, HALF + 1):
            k = (dy + HALF) * WIN + (dx + HALF)
            emit_pair(xn_dy, dy, dx, k)
        return 0

    lax.fori_loop(1, HALF + 1, dy_body, 0)
    sim_ref[K2:, :] = jnp.full((K2P - K2, P), NEG_INF, jnp.float32)


def _sc_kernel(sim_hbm, xt_hbm, agg_hbm,
               sblk, wj2, gidxa, gidxb, rows, accb, sem):
    wid = lax.axis_index("s") * 2 + lax.axis_index("c")

    @pl.when(wid < NWORK)
    def _():
        base = wid * PXW
        pltpu.sync_copy(sim_hbm.at[:, pl.ds(base, PXW)], sblk)
        iotaf = lax.iota(jnp.int32, LANES).astype(jnp.float32)

        for g in range(NGRP):
            lo = g * LANES

            def body(k, t):
                vals, idxs = t
                v = sblk[k, pl.ds(lo, LANES)]
                kv = jnp.zeros((LANES,), jnp.float32) + k.astype(jnp.float32)
                nv, ni = [], []
                for i in range(KSEL):
                    if i == 0:
                        keep = vals[0] >= v
                        nv.append(jnp.maximum(vals[0], v))
                        ni.append(jnp.where(keep, idxs[0], kv))
                    else:
                        m = jnp.minimum(vals[i - 1], v)
                        pm = jnp.where(v <= vals[i - 1], kv, idxs[i - 1])
                        keep = vals[i] >= m
                        nv.append(jnp.maximum(vals[i], m))
                        ni.append(jnp.where(keep, idxs[i], pm))
                return (tuple(nv), tuple(ni))

            init = (tuple(jnp.full((LANES,), NEG_INF, jnp.float32)
                          for _ in range(KSEL)),
                    tuple(jnp.zeros((LANES,), jnp.float32)
                          for _ in range(KSEL)))
            vals, idxs = lax.fori_loop(0, K2, body, init)

            den = (jnp.exp(vals[0]) + jnp.exp(vals[1]) + jnp.exp(vals[2])
                   + jnp.exp(vals[3]) + jnp.exp(vals[4]) + jnp.exp(vals[5])
                   + jnp.exp(vals[6]) + jnp.exp(vals[7]))
            rden = 1.0 / den
            pbasef = jnp.float32(base + lo) + iotaf

            for j in range(KSEL):
                wj2[pl.ds(j * PXW + lo, LANES)] = jnp.exp(vals[j]) * rden
                kf = idxs[j]
                dxf = jnp.mod(kf, float(WIN))
                dyf = (kf - dxf) * (1.0 / WIN)
                offf = (dyf - HALF) * W + (dxf - HALF)
                gi = (pbasef + offf).astype(jnp.int32)
                if g < NGRP // 2:
                    gidxa[pl.ds(j * (PXW // 2) + lo, LANES)] = gi
                else:
                    gidxb[pl.ds(j * (PXW // 2) + lo - PXW // 2, LANES)] = gi

        for c, gref in enumerate((gidxa, gidxb)):
            pltpu.async_copy(xt_hbm.at[gref], rows, sem).wait()

            def bodyg(g, _):
                lo_t = (c * (NGRP // 2) + g) * LANES
                wv = [wj2[pl.ds(j * PXW + lo_t, LANES)]
                      for j in range(KSEL)]
                for pxi in range(LANES):
                    accs = [jnp.zeros((LANES,), jnp.float32)
                            for _ in range(C // LANES)]
                    for j in range(KSEL):
                        ws = wv[j][pxi]
                        r = j * (PXW // 2) + g * LANES + pxi
                        for i in range(C // LANES):
                            seg = rows[r, pl.ds(i * LANES, LANES)]
                            accs[i] = accs[i] + ws * seg
                    for i in range(C // LANES):
                        accb[lo_t + pxi, pl.ds(i * LANES, LANES)] = accs[i]
                return 0

            lax.fori_loop(0, NGRP // 2, bodyg, 0)

        pltpu.sync_copy(accb, agg_hbm.at[pl.ds(base, PXW)])


def _sc_stage(sim, xt_pad):
    mesh = plsc.VectorSubcoreMesh(core_axis_name="c", subcore_axis_name="s")
    return pl.kernel(
        _sc_kernel,
        out_type=jax.ShapeDtypeStruct((P, C), jnp.float32),
        mesh=mesh,
        scratch_types=[
            pltpu.VMEM((K2P, PXW), jnp.float32),
            pltpu.VMEM((PXW * KSEL,), jnp.float32),
            pltpu.VMEM((CHUNK,), jnp.int32),
            pltpu.VMEM((CHUNK,), jnp.int32),
            pltpu.VMEM((CHUNK, 128), jnp.float32),
            pltpu.VMEM((PXW, C), jnp.float32),
            pltpu.SemaphoreType.DMA,
        ],
    )(sim, xt_pad)


def _ffn_kernel(xt_ref, agg_ref, lnw_ref, lnb_ref, w1t_ref, b1_ref,
                w2t_ref, b2_ref, out_ref):
    XT = xt_ref[...]
    mu = jnp.sum(XT, axis=1, keepdims=True) * (1.0 / C)
    xc = XT - mu
    var = jnp.sum(xc * xc, axis=1, keepdims=True) * (1.0 / C)
    xln = xc / jnp.sqrt(var + 1e-5) * lnw_ref[...] + lnb_ref[...]

    enh = agg_ref[...] + xln

    h = jnp.dot(enh, w1t_ref[...], preferred_element_type=jnp.float32)
    h = jnp.maximum(h + b1_ref[...], 0.0)
    ffn = jnp.dot(h, w2t_ref[...], preferred_element_type=jnp.float32)
    out_ref[...] = enh + ffn + b2_ref[...]


@jax.jit
def kernel(x, ln_w, ln_b, w1, b1, w2, b2):
    xf = x.reshape(C, P)
    xt = xf.T
    xt_pad = jnp.pad(xt, ((0, 0), (0, 128 - C)))
    sim = pl.pallas_call(
        _sim_kernel,
        out_shape=jax.ShapeDtypeStruct((K2P, P), jnp.float32),
    )(xf)
    agg = _sc_stage(sim, xt_pad)
    out = pl.pallas_call(
        _ffn_kernel,
        out_shape=jax.ShapeDtypeStruct((P, C), jnp.float32),
    )(xt, agg, ln_w.reshape(1, C), ln_b.reshape(1, C),
      w1.T, b1.reshape(1, 2 * C), w2.T, b2.reshape(1, C))
    return out.T.reshape(1, C, H, W)


# speedup vs baseline: 2.5700x; 1.1250x over previous
import functools
import math

import jax
import jax.numpy as jnp
from jax import lax
from jax.experimental import pallas as pl
from jax.experimental.pallas import tpu as pltpu
from jax.experimental.pallas import tpu_sc as plsc

C = 96
H = 48
W = 48
P = H * W
WIN = 13
HALF = WIN // 2
K2 = WIN * WIN
K2P = 176
KSEL = 8
NEG_INF = float("-inf")

LANES = 16
NHALF = 16
PXA = 80
PXB = 64
NG = 5
SBW = 256
GA = 3 * 8 * LANES
GB = 2 * 8 * LANES


def _sim_kernel(x_ref, sim_ref):
    X = x_ref[...]
    n2 = jnp.sum(X * X, axis=0, keepdims=True)
    norm = jnp.maximum(jnp.sqrt(n2), 1e-12)
    XN = X / norm

    p = lax.broadcasted_iota(jnp.int32, (1, P), 1)
    px = p % W
    py = p // W

    def valid_mask(dy, dx):
        nx = px + dx
        ny = py + dy
        return (nx >= 0) & (nx < W) & (ny >= 0) & (ny < H)

    def emit_pair(xn_dy, dy, dx, k):
        sh = pltpu.roll(xn_dy, (-dx) % P, axis=1)
        s_raw = jnp.sum(XN * sh, axis=0, keepdims=True)
        sim_ref[pl.ds(k, 1), :] = jnp.where(valid_mask(dy, dx), s_raw,
                                            NEG_INF)
        if isinstance(k, int) and k == K2 // 2:
            return
        off = dy * W + dx
        s_neg = pltpu.roll(s_raw, jnp.mod(off, P), axis=1)
        sim_ref[pl.ds(K2 - 1 - k, 1), :] = jnp.where(valid_mask(-dy, -dx),
                                                     s_neg, NEG_INF)

    for dx in range(0, HALF + 1):
        emit_pair(XN, 0, dx, (K2 // 2) + dx)

    def dy_body(dy, _):
        xn_dy = pltpu.roll(XN, jnp.mod(-dy * W, P), axis=1)
        for dx in range(-HALF, HALF + 1):
            k = (dy + HALF) * WIN + (dx + HALF)
            emit_pair(xn_dy, dy, dx, k)
        return 0

    lax.fori_loop(1, HALF + 1, dy_body, 0)
    sim_ref[K2:, :] = jnp.full((K2P - K2, P), NEG_INF, jnp.float32)


def _sc_kernel(sim_hbm, xt_hbm, agg_hbm,
               sblk, wj2, gidxa, gidxb, rows, accb, sem):
    wid = lax.axis_index("s") * 2 + lax.axis_index("c")
    is_a = wid < NHALF
    base = jnp.where(is_a, wid * PXA, NHALF * PXA + (wid - NHALF) * PXB)
    albase = jnp.minimum((base // 128) * 128, P - SBW)
    head = base - albase
    iotaf = lax.iota(jnp.int32, LANES).astype(jnp.float32)

    pltpu.sync_copy(sim_hbm.at[:, pl.ds(albase, SBW)], sblk)

    goff_last = jnp.where(is_a, 64, 48)
    for g in range(NG):
        goff = g * LANES if g < NG - 1 else goff_last
        lo = head + goff

        def body(k, t):
            v = sblk[k, pl.ds(lo, LANES)]
            t1, t2, t3, t4, t5, t6, t7, t8 = t
            n8 = jnp.maximum(t8, jnp.minimum(t7, v))
            n7 = jnp.maximum(t7, jnp.minimum(t6, v))
            n6 = jnp.maximum(t6, jnp.minimum(t5, v))
            n5 = jnp.maximum(t5, jnp.minimum(t4, v))
            n4 = jnp.maximum(t4, jnp.minimum(t3, v))
            n3 = jnp.maximum(t3, jnp.minimum(t2, v))
            n2 = jnp.maximum(t2, jnp.minimum(t1, v))
            n1 = jnp.maximum(t1, v)
            return (n1, n2, n3, n4, n5, n6, n7, n8)

        init = tuple(jnp.full((LANES,), NEG_INF, jnp.float32)
                     for _ in range(KSEL))
        vals = lax.fori_loop(0, K2, body, init)

        def match(k, idxs):
            v = sblk[k, pl.ds(lo, LANES)]
            kv = jnp.zeros((LANES,), jnp.float32) + k.astype(jnp.float32)
            return tuple(jnp.where(v == vals[j], kv, idxs[j])
                         for j in range(KSEL))

        idxs = lax.fori_loop(0, K2, match,
                             tuple(jnp.zeros((LANES,), jnp.float32)
                                   for _ in range(KSEL)))

        den = (jnp.exp(vals[0]) + jnp.exp(vals[1]) + jnp.exp(vals[2])
               + jnp.exp(vals[3]) + jnp.exp(vals[4]) + jnp.exp(vals[5])
               + jnp.exp(vals[6]) + jnp.exp(vals[7]))
        rden = 1.0 / den
        pbasef = (base + goff).astype(jnp.float32) + iotaf

        for j in range(KSEL):
            sl = g * (LANES * KSEL) + j * LANES
            wj2[pl.ds(sl, LANES)] = jnp.exp(vals[j]) * rden
            kf = idxs[j]
            dxf = jnp.mod(kf, float(WIN))
            dyf = (kf - dxf) * (1.0 / WIN)
            offf = (dyf - HALF) * W + (dxf - HALF)
            gi = (pbasef + offf).astype(jnp.int32)
            if g < 3:
                gidxa[pl.ds(sl, LANES)] = gi
            else:
                gidxb[pl.ds(sl - GA, LANES)] = gi

    def make_acc(c, nchunk):
        def acc_body(g, _):
            gl = c * 3 + g
            lo = jnp.where(gl == NG - 1, goff_last, gl * LANES)
            wv = [wj2[pl.ds(gl * (LANES * KSEL) + j * LANES, LANES)]
                  for j in range(KSEL)]
            for pxi in range(LANES):
                accs = [jnp.zeros((LANES,), jnp.float32)
                        for _ in range(C // LANES)]
                for j in range(KSEL):
                    ws = wv[j][pxi]
                    r = g * (LANES * KSEL) + j * LANES + pxi
                    for i in range(C // LANES):
                        seg = rows[r, pl.ds(i * LANES, LANES)]
                        accs[i] = accs[i] + ws * seg
                for i in range(C // LANES):
                    accb[lo + pxi, pl.ds(i * LANES, LANES)] = accs[i]
            return 0
        return acc_body

    pltpu.async_copy(xt_hbm.at[gidxa], rows.at[pl.ds(0, GA)], sem).wait()
    lax.fori_loop(0, 3, make_acc(0, GA), 0)
    pltpu.async_copy(xt_hbm.at[gidxb], rows.at[pl.ds(0, GB)], sem).wait()
    lax.fori_loop(0, 2, make_acc(1, GB), 0)

    @pl.when(is_a)
    def _():
        pltpu.sync_copy(accb.at[pl.ds(0, PXA)], agg_hbm.at[pl.ds(base, PXA)])

    @pl.when(jnp.logical_not(is_a))
    def _():
        pltpu.sync_copy(accb.at[pl.ds(0, PXB)], agg_hbm.at[pl.ds(base, PXB)])


def _sc_stage(sim, xt_pad):
    mesh = plsc.VectorSubcoreMesh(core_axis_name="c", subcore_axis_name="s")
    return pl.kernel(
        _sc_kernel,
        out_type=jax.ShapeDtypeStruct((P, C), jnp.float32),
        mesh=mesh,
        scratch_types=[
            pltpu.VMEM((K2P, SBW), jnp.float32),
            pltpu.VMEM((NG * LANES * KSEL,), jnp.float32),
            pltpu.VMEM((GA,), jnp.int32),
            pltpu.VMEM((GB,), jnp.int32),
            pltpu.VMEM((GA, 128), jnp.float32),
            pltpu.VMEM((NG * LANES, C), jnp.float32),
            pltpu.SemaphoreType.DMA,
        ],
    )(sim, xt_pad)


def _ffn_kernel(xt_ref, agg_ref, lnw_ref, lnb_ref, w1t_ref, b1_ref,
                w2t_ref, b2_ref, out_ref):
    XT = xt_ref[...]
    mu = jnp.sum(XT, axis=1, keepdims=True) * (1.0 / C)
    xc = XT - mu
    var = jnp.sum(xc * xc, axis=1, keepdims=True) * (1.0 / C)
    xln = xc / jnp.sqrt(var + 1e-5) * lnw_ref[...] + lnb_ref[...]

    enh = agg_ref[...] + xln

    h = jnp.dot(enh, w1t_ref[...], preferred_element_type=jnp.float32)
    h = jnp.maximum(h + b1_ref[...], 0.0)
    ffn = jnp.dot(h, w2t_ref[...], preferred_element_type=jnp.float32)
    out_ref[...] = enh + ffn + b2_ref[...]


@jax.jit
def kernel(x, ln_w, ln_b, w1, b1, w2, b2):
    xf = x.reshape(C, P)
    xt = xf.T
    xt_pad = jnp.pad(xt, ((0, 0), (0, 128 - C)))
    sim = pl.pallas_call(
        _sim_kernel,
        out_shape=jax.ShapeDtypeStruct((K2P, P), jnp.float32),
    )(xf)
    agg = _sc_stage(sim, xt_pad)
    out = pl.pallas_call(
        _ffn_kernel,
        out_shape=jax.ShapeDtypeStruct((P, C), jnp.float32),
    )(xt, agg, ln_w.reshape(1, C), ln_b.reshape(1, C),
      w1.T, b1.reshape(1, 2 * C), w2.T, b2.reshape(1, C))
    return out.T.reshape(1, C, H, W)
